# baseline (speedup 1.0000x reference)
import functools
import jax
import jax.numpy as jnp
from jax.experimental import pallas as pl
from jax.experimental.pallas import tpu as pltpu

C_EDGE = 256
B_SEG = 256
GCHUNK = 128


def _round_up(a, b):
    return (a + b - 1) // b * b


def _sc_gather(table, idx):
    from jax.experimental.pallas import tpu_sc as plsc

    info = plsc.get_sparse_core_info()
    nw = info.num_cores * info.num_subcores
    maxp = idx.shape[0]
    d = table.shape[1]
    per_w = maxp // nw
    n_chunks = per_w // GCHUNK
    mesh = plsc.VectorSubcoreMesh(core_axis_name="c", subcore_axis_name="s")

    @functools.partial(
        pl.kernel, mesh=mesh,
        out_type=jax.ShapeDtypeStruct((maxp, d), jnp.float32),
        scratch_types=[
            pltpu.VMEM((GCHUNK,), jnp.int32),
            pltpu.VMEM((GCHUNK, d), jnp.float32),
            pltpu.SemaphoreType.DMA,
        ],
    )
    def k(table_hbm, idx_hbm, out_hbm, idx_v, rows_v, sem):
        wid = jax.lax.axis_index("s") * info.num_cores + jax.lax.axis_index("c")
        base = wid * per_w
        for i in range(n_chunks):
            off = base + i * GCHUNK
            pltpu.sync_copy(idx_hbm.at[pl.ds(off, GCHUNK)], idx_v)
            pltpu.async_copy(table_hbm.at[idx_v], rows_v, sem).wait()
            pltpu.sync_copy(rows_v, out_hbm.at[pl.ds(off, GCHUNK)])

    return k(table, idx)


def _segsum_body(ob_ref, fv_ref, seg_ref, xg_ref, acc_ref, cnt_ref):
    k = pl.program_id(0)
    base = ob_ref[k] * B_SEG
    seg = seg_ref[k, :]
    iota = jax.lax.broadcasted_iota(jnp.int32, (B_SEG, C_EDGE), 0)
    lt = (seg[None, :] == base + iota).astype(jnp.float32)
    contrib = jnp.dot(lt, xg_ref[...], preferred_element_type=jnp.float32)
    ccontrib = jnp.broadcast_to(jnp.sum(lt, axis=1)[None, None, :], (1, 8, B_SEG))
    fv = fv_ref[k]

    @pl.when(fv == 1)
    def _():
        acc_ref[...] = contrib
        cnt_ref[...] = ccontrib

    @pl.when(fv == 0)
    def _():
        acc_ref[...] += contrib
        cnt_ref[...] += ccontrib


def _segsum(xg, seg_pad, outblk, firstvisit, nblocks):
    kk = xg.shape[0] // C_EDGE
    d = xg.shape[1]
    grid_spec = pltpu.PrefetchScalarGridSpec(
        num_scalar_prefetch=2,
        grid=(kk,),
        in_specs=[
            pl.BlockSpec((kk, C_EDGE), lambda k, ob, fv: (0, 0)),
            pl.BlockSpec((C_EDGE, d), lambda k, ob, fv: (k, 0)),
        ],
        out_specs=[
            pl.BlockSpec((B_SEG, d), lambda k, ob, fv: (ob[k], 0)),
            pl.BlockSpec((1, 8, B_SEG), lambda k, ob, fv: (ob[k], 0, 0)),
        ],
    )
    return pl.pallas_call(
        _segsum_body,
        grid_spec=grid_spec,
        out_shape=[
            jax.ShapeDtypeStruct((nblocks * B_SEG, d), jnp.float32),
            jax.ShapeDtypeStruct((nblocks, 8, B_SEG), jnp.float32),
        ],
    )(outblk, firstvisit, seg_pad.reshape(kk, C_EDGE), xg)


def _dense_body(h_ref, a_ref, c_ref, wrel_ref, wroot_ref, b_ref, out_ref, *, nrel):
    acc = jnp.dot(h_ref[...], wroot_ref[...], preferred_element_type=jnp.float32)
    acc += b_ref[0, :][None, :]
    for r in range(nrel):
        ar = a_ref[r] / jnp.maximum(c_ref[:, r], 1.0)[:, None]
        acc += jnp.dot(ar, wrel_ref[r], preferred_element_type=jnp.float32)
    out_ref[...] = jnp.maximum(acc, 0.0)


def _dense_final_body(h_ref, a_ref, c_ref, wrel_ref, wroot_ref, b_ref,
                      wlin_ref, blin_ref, out_ref, *, nrel):
    acc = jnp.dot(h_ref[...], wroot_ref[...], preferred_element_type=jnp.float32)
    acc += b_ref[0, :][None, :]
    for r in range(nrel):
        ar = a_ref[r] / jnp.maximum(c_ref[:, r], 1.0)[:, None]
        acc += jnp.dot(ar, wrel_ref[r], preferred_element_type=jnp.float32)
    acc = jnp.maximum(acc, 0.0)
    out_ref[...] = (jnp.dot(acc, wlin_ref[...], preferred_element_type=jnp.float32)
                    + blin_ref[0, :][None, :])


def _dense(h, a, cnt, w_rel, w_root, b, wlin=None, blin=None, block_n=400):
    n, din = h.shape
    nrel = w_rel.shape[0]
    dh = w_rel.shape[2]
    grid = (n // block_n,)
    in_specs = [
        pl.BlockSpec((block_n, din), lambda i: (i, 0)),
        pl.BlockSpec((nrel, block_n, din), lambda i: (0, i, 0)),
        pl.BlockSpec((block_n, nrel), lambda i: (i, 0)),
        pl.BlockSpec((nrel, din, dh), lambda i: (0, 0, 0)),
        pl.BlockSpec((din, dh), lambda i: (0, 0)),
        pl.BlockSpec((1, dh), lambda i: (0, 0)),
    ]
    args = [h, a, cnt.T, w_rel, w_root, b.reshape(1, -1)]
    if wlin is None:
        body = functools.partial(_dense_body, nrel=nrel)
        dout = dh
    else:
        dout = wlin.shape[1]
        in_specs += [
            pl.BlockSpec((dh, dout), lambda i: (0, 0)),
            pl.BlockSpec((1, dout), lambda i: (0, 0)),
        ]
        args += [wlin, blin.reshape(1, -1)]
        body = functools.partial(_dense_final_body, nrel=nrel)
    return pl.pallas_call(
        body,
        grid=grid,
        in_specs=in_specs,
        out_specs=pl.BlockSpec((block_n, dout), lambda i: (i, 0)),
        out_shape=jax.ShapeDtypeStruct((n, dout), jnp.float32),
    )(*args)


def kernel(x, edge_index, edge_type, W1_rel, W1_root, b1, W2_rel, W2_root, b2,
           Wlin, blin):
    n = x.shape[0]
    e = edge_index.shape[1]
    nrel = W1_rel.shape[0]
    s_total = nrel * n
    nblocks = _round_up(s_total, B_SEG) // B_SEG

    src = edge_index[0]
    dst = edge_index[1]
    seg = edge_type * n + dst
    order = jnp.argsort(seg)
    seg_s = seg[order]
    src_s = src[order]
    blk_e = seg_s // B_SEG
    off_blk = jnp.searchsorted(blk_e, jnp.arange(nblocks + 1, dtype=jnp.int32),
                               side="left").astype(jnp.int32)
    e_b = off_blk[1:] - off_blk[:-1]
    padded_b = jnp.maximum((e_b + C_EDGE - 1) // C_EDGE, 1) * C_EDGE
    pad_off = jnp.concatenate([jnp.zeros((1,), jnp.int32),
                               jnp.cumsum(padded_b).astype(jnp.int32)])
    maxp = _round_up(e + nblocks * C_EDGE, 32 * GCHUNK)
    pos = pad_off[blk_e] + (jnp.arange(e, dtype=jnp.int32) - off_blk[blk_e])
    src_pad = jnp.zeros((maxp,), jnp.int32).at[pos].set(src_s)
    seg_pad = jnp.full((maxp,), -1, jnp.int32).at[pos].set(seg_s)
    kk = maxp // C_EDGE
    chunk_starts = jnp.arange(kk, dtype=jnp.int32) * C_EDGE
    outblk = jnp.clip(
        jnp.searchsorted(pad_off, chunk_starts, side="right") - 1,
        0, nblocks - 1).astype(jnp.int32)
    firstvisit = jnp.concatenate([
        jnp.ones((1,), jnp.int32),
        (outblk[1:] != outblk[:-1]).astype(jnp.int32)])

    def layer(h, w_rel, w_root, b, wlin=None, blin_=None):
        xg = _sc_gather(h, src_pad)
        a_blk, c_blk = _segsum(xg, seg_pad, outblk, firstvisit, nblocks)
        a = a_blk[:s_total].reshape(nrel, n, h.shape[1])
        cnt = c_blk[:, 0, :].reshape(-1)[:s_total].reshape(nrel, n)
        return _dense(h, a, cnt, w_rel, w_root, b, wlin, blin_)

    h1 = layer(x, W1_rel, W1_root, b1)
    return layer(h1, W2_rel, W2_root, b2, Wlin, blin)
